# baseline (speedup 1.0000x reference)
import jax
import jax.numpy as jnp
from jax import lax
from jax.experimental import pallas as pl
from jax.experimental.pallas import tpu as pltpu
from jax.experimental.pallas import tpu_sc as plsc

C = 96
H = 512
W = 512
FH = 128
FW = 128
L = FH * FW
NW = 32
PR_PER_W = FH // NW
ROWS_PER_W = 4 * PR_PER_W
LW = PR_PER_W * FW
CH_BLK = 4
NB = 2
GRP = CH_BLK * NB


def _slab_src(x_hbm, wid, c):
    return x_hbm.at[pl.ds(c, CH_BLK), pl.ds(wid * ROWS_PER_W, ROWS_PER_W), :]


def _body(x_hbm, s_hbm, out_hbm, sbuf, rbuf, cbuf, xb0, xb1, obuf, load_sem):
    xbufs = (xb0, xb1)
    cid = lax.axis_index("c")
    sid = lax.axis_index("s")
    wid = sid * 2 + cid
    base_l = wid * LW

    for b in range(NB):
        pltpu.async_copy(_slab_src(x_hbm, wid, b * CH_BLK), xbufs[b], load_sem)

    pltpu.sync_copy(s_hbm.at[pl.ds(base_l, LW)], sbuf)
    lane4 = lax.iota(jnp.int32, 16) * 4
    zero = lane4 * 0
    for i in range(LW // 16):
        s = sbuf[pl.ds(i * 16, 16)]
        rbuf[pl.ds(i * 16, 16)] = (s >> 2) + ((i * 16) // FW) * 4
        cbuf[pl.ds(i * 16, 16)] = (s & 3) + lane4 + ((i * 16) % FW) * 4

    def group(g, carry):
        for b in range(NB):
            c0 = g * GRP + b * CH_BLK
            pltpu.make_async_copy(
                _slab_src(x_hbm, wid, c0), xbufs[b], load_sem
            ).wait()
            for cc in range(CH_BLK):
                cvec = zero + cc
                for i in range(LW // 16):
                    row = rbuf[pl.ds(i * 16, 16)]
                    col = cbuf[pl.ds(i * 16, 16)]
                    obuf[c0 + cc, pl.ds(i * 16, 16)] = plsc.load_gather(
                        xbufs[b], [cvec, row, col]
                    )

            @pl.when(c0 + GRP < C)
            def _():
                pltpu.async_copy(
                    _slab_src(x_hbm, wid, c0 + GRP), xbufs[b], load_sem
                )

        return carry

    lax.fori_loop(0, C // GRP, group, 0)
    pltpu.sync_copy(obuf, out_hbm.at[:, pl.ds(base_l, LW)])


@jax.jit
def _run(xr, sidx):
    mesh = plsc.VectorSubcoreMesh(core_axis_name="c", subcore_axis_name="s")
    kfn = pl.kernel(
        _body,
        out_type=jax.ShapeDtypeStruct((C, L), jnp.float32),
        mesh=mesh,
        scratch_types=[
            pltpu.VMEM((LW,), jnp.int32),
            pltpu.VMEM((LW,), jnp.int32),
            pltpu.VMEM((LW,), jnp.int32),
            pltpu.VMEM((CH_BLK, ROWS_PER_W, W), jnp.float32),
            pltpu.VMEM((CH_BLK, ROWS_PER_W, W), jnp.float32),
            pltpu.VMEM((C, LW), jnp.float32),
            pltpu.SemaphoreType.DMA,
        ],
        compiler_params=pltpu.CompilerParams(needs_layout_passes=False),
    )
    return kfn(xr, sidx)


def kernel(x):
    b, c, h, w = x.shape
    sidx = jax.random.randint(jax.random.key(42), (L,), 0, 16, jnp.int32)
    xr = x.reshape(C, H, W)
    out = _run(xr, sidx)
    return out.reshape(1, C, L)


# speedup vs baseline: 1.0890x; 1.0890x over previous
import jax
import jax.numpy as jnp
import numpy as np
from jax import lax
from jax.experimental import pallas as pl
from jax.experimental.pallas import tpu as pltpu
from jax.experimental.pallas import tpu_sc as plsc

C = 96
H = 512
W = 512
FH = 128
FW = 128
L = FH * FW
NW = 32
PR_PER_W = FH // NW
ROWS_PER_W = 4 * PR_PER_W
LW = PR_PER_W * FW
NB = 8

_SIDX = np.asarray(
    jax.random.randint(jax.random.key(42), (L,), 0, 16, jnp.int32))


def _slab_src(x_hbm, wid, c):
    return x_hbm.at[pl.ds(c * H + wid * ROWS_PER_W, ROWS_PER_W), :]


def _body(x_hbm, s_hbm, out_hbm, sbuf, rbuf, cbuf, xbufs, obuf, load_sem):
    cid = lax.axis_index("c")
    sid = lax.axis_index("s")
    wid = sid * 2 + cid
    base_l = wid * LW

    for b in range(NB):
        pltpu.async_copy(_slab_src(x_hbm, wid, b), xbufs[b], load_sem)

    pltpu.sync_copy(s_hbm.at[pl.ds(base_l, LW)], sbuf)
    lane4 = lax.iota(jnp.int32, 16) * 4
    for i in range(LW // 16):
        s = sbuf[pl.ds(i * 16, 16)]
        rbuf[pl.ds(i * 16, 16)] = (s >> 2) + ((i * 16) // FW) * 4
        cbuf[pl.ds(i * 16, 16)] = (s & 3) + lane4 + ((i * 16) % FW) * 4

    def group(g, carry):
        for b in range(NB):
            c = g * NB + b
            pltpu.make_async_copy(
                _slab_src(x_hbm, wid, c), xbufs[b], load_sem
            ).wait()
            for i in range(LW // 16):
                row = rbuf[pl.ds(i * 16, 16)]
                col = cbuf[pl.ds(i * 16, 16)]
                obuf[c, pl.ds(i * 16, 16)] = plsc.load_gather(
                    xbufs[b], [row, col]
                )

            @pl.when(c + NB < C)
            def _():
                pltpu.async_copy(
                    _slab_src(x_hbm, wid, c + NB), xbufs[b], load_sem
                )

        return carry

    lax.fori_loop(0, C // NB, group, 0)
    pltpu.sync_copy(obuf, out_hbm.at[:, pl.ds(base_l, LW)])


def _body_wrap(x_hbm, s_hbm, out_hbm, sbuf, rbuf, cbuf, *rest):
    xbufs = rest[:NB]
    obuf, load_sem = rest[NB:]
    _body(x_hbm, s_hbm, out_hbm, sbuf, rbuf, cbuf, xbufs, obuf, load_sem)


@jax.jit
def _run(xr, sidx):
    mesh = plsc.VectorSubcoreMesh(core_axis_name="c", subcore_axis_name="s")
    kfn = pl.kernel(
        _body_wrap,
        out_type=jax.ShapeDtypeStruct((C, L), jnp.float32),
        mesh=mesh,
        scratch_types=[
            pltpu.VMEM((LW,), jnp.int32),
            pltpu.VMEM((LW,), jnp.int32),
            pltpu.VMEM((LW,), jnp.int32),
        ] + [
            pltpu.VMEM((ROWS_PER_W, W), jnp.float32)
            for _ in range(NB)
        ] + [
            pltpu.VMEM((C, LW), jnp.float32),
            pltpu.SemaphoreType.DMA,
        ],
        compiler_params=pltpu.CompilerParams(needs_layout_passes=False),
    )
    return kfn(xr, sidx)


def kernel(x):
    b, c, h, w = x.shape
    sidx = jnp.asarray(_SIDX)
    xr = x.reshape(C * H, W)
    out = _run(xr, sidx)
    return out.reshape(1, C, L)


# speedup vs baseline: 1.3060x; 1.1993x over previous
import jax
import jax.numpy as jnp
import numpy as np
from jax import lax
from jax.experimental import pallas as pl
from jax.experimental.pallas import tpu as pltpu
from jax.experimental.pallas import tpu_sc as plsc

C = 96
H = 512
W = 512
FH = 128
FW = 128
L = FH * FW
NW = 32
PR_PER_W = FH // NW
ROWS_PER_W = 4 * PR_PER_W
LW = PR_PER_W * FW
NB = 4

_SIDX = np.asarray(
    jax.random.randint(jax.random.key(42), (L,), 0, 16, jnp.int32))


def _slab_src(x_hbm, wid, c):
    return x_hbm.at[pl.ds(c * H + wid * ROWS_PER_W, ROWS_PER_W), :]


def _body(x_hbm, s_hbm, out_hbm, sbuf, rbuf, cbuf, xbufs, obuf, load_sem):
    cid = lax.axis_index("c")
    sid = lax.axis_index("s")
    wid = sid * 2 + cid
    base_l = wid * LW

    for b in range(NB):
        pltpu.async_copy(_slab_src(x_hbm, wid, b), xbufs[b], load_sem)

    pltpu.sync_copy(s_hbm.at[pl.ds(base_l, LW)], sbuf)
    lane4 = lax.iota(jnp.int32, 16) * 4
    for i in range(LW // 16):
        s = sbuf[pl.ds(i * 16, 16)]
        rbuf[pl.ds(i * 16, 16)] = (s >> 2) + ((i * 16) // FW) * 4
        cbuf[pl.ds(i * 16, 16)] = (s & 3) + lane4 + ((i * 16) % FW) * 4

    def group(g, carry):
        for b in range(NB):
            c = g * NB + b
            pltpu.make_async_copy(
                _slab_src(x_hbm, wid, c), xbufs[b], load_sem
            ).wait()
            for i in range(LW // 16):
                row = rbuf[pl.ds(i * 16, 16)]
                col = cbuf[pl.ds(i * 16, 16)]
                obuf[c, pl.ds(i * 16, 16)] = plsc.load_gather(
                    xbufs[b], [row, col]
                )

            @pl.when(c + NB < C)
            def _():
                pltpu.async_copy(
                    _slab_src(x_hbm, wid, c + NB), xbufs[b], load_sem
                )

        return carry

    lax.fori_loop(0, C // NB, group, 0)
    pltpu.sync_copy(obuf, out_hbm.at[:, pl.ds(base_l, LW)])


def _body_wrap(x_hbm, s_hbm, out_hbm, sbuf, rbuf, cbuf, *rest):
    xbufs = rest[:NB]
    obuf, load_sem = rest[NB:]
    _body(x_hbm, s_hbm, out_hbm, sbuf, rbuf, cbuf, xbufs, obuf, load_sem)


@jax.jit
def _run(xr, sidx):
    mesh = plsc.VectorSubcoreMesh(core_axis_name="c", subcore_axis_name="s")
    kfn = pl.kernel(
        _body_wrap,
        out_type=jax.ShapeDtypeStruct((C, L), jnp.float32),
        mesh=mesh,
        scratch_types=[
            pltpu.VMEM((LW,), jnp.int32),
            pltpu.VMEM((LW,), jnp.int32),
            pltpu.VMEM((LW,), jnp.int32),
        ] + [
            pltpu.VMEM((ROWS_PER_W, W), jnp.float32)
            for _ in range(NB)
        ] + [
            pltpu.VMEM((C, LW), jnp.float32),
            pltpu.SemaphoreType.DMA,
        ],
        compiler_params=pltpu.CompilerParams(needs_layout_passes=False),
    )
    return kfn(xr, sidx)


def kernel(x):
    b, c, h, w = x.shape
    sidx = jnp.asarray(_SIDX)
    xr = x.reshape(C * H, W)
    out = _run(xr, sidx)
    return out.reshape(1, C, L)


# speedup vs baseline: 1.4431x; 1.1050x over previous
import jax
import jax.numpy as jnp
import numpy as np
from jax import lax
from jax.experimental import pallas as pl
from jax.experimental.pallas import tpu as pltpu
from jax.experimental.pallas import tpu_sc as plsc

C = 96
H = 512
W = 512
FH = 128
FW = 128
L = FH * FW
NW = 32
PR_PER_W = FH // NW
ROWS_PER_W = 4 * PR_PER_W
LW = PR_PER_W * FW
CG = 4
NB = 2 * CG

_SIDX = np.asarray(
    jax.random.randint(jax.random.key(42), (L,), 0, 16, jnp.int32))


def _slab_src(x_hbm, wid, c):
    return x_hbm.at[pl.ds(c * H + wid * ROWS_PER_W, ROWS_PER_W), :]


def _body(x_hbm, s_hbm, out_hbm, sbuf, ibuf, xbufs, obuf, load_sem):
    cid = lax.axis_index("c")
    sid = lax.axis_index("s")
    wid = sid * 2 + cid
    base_l = wid * LW

    for b in range(NB):
        pltpu.async_copy(_slab_src(x_hbm, wid, b), xbufs[b], load_sem)

    pltpu.sync_copy(s_hbm.at[pl.ds(base_l, LW)], sbuf)
    lane4 = lax.iota(jnp.int32, 16) * 4
    for i in range(LW // 16):
        s = sbuf[pl.ds(i * 16, 16)]
        row = (s >> 2) + ((i * 16) // FW) * 4
        col = (s & 3) + lane4 + ((i * 16) % FW) * 4
        ibuf[pl.ds(i * 16, 16)] = (row << 9) | col

    def group(g, carry):
        for half in range(NB // CG):
            c0 = g * NB + half * CG
            for k in range(CG):
                pltpu.make_async_copy(
                    _slab_src(x_hbm, wid, c0 + k), xbufs[half * CG + k],
                    load_sem,
                ).wait()
            for i in range(LW // 16):
                p = ibuf[pl.ds(i * 16, 16)]
                row = p >> 9
                col = p & (W - 1)
                for k in range(CG):
                    obuf[c0 + k, pl.ds(i * 16, 16)] = plsc.load_gather(
                        xbufs[half * CG + k], [row, col]
                    )

            @pl.when(c0 + NB + CG <= C)
            def _():
                for k in range(CG):
                    pltpu.async_copy(
                        _slab_src(x_hbm, wid, c0 + NB + k),
                        xbufs[half * CG + k], load_sem,
                    )

        return carry

    lax.fori_loop(0, C // NB, group, 0)
    pltpu.sync_copy(obuf, out_hbm.at[:, pl.ds(base_l, LW)])


def _body_wrap(x_hbm, s_hbm, out_hbm, sbuf, ibuf, *rest):
    xbufs = rest[:NB]
    obuf, load_sem = rest[NB:]
    _body(x_hbm, s_hbm, out_hbm, sbuf, ibuf, xbufs, obuf, load_sem)


@jax.jit
def _run(xr, sidx):
    mesh = plsc.VectorSubcoreMesh(core_axis_name="c", subcore_axis_name="s")
    kfn = pl.kernel(
        _body_wrap,
        out_type=jax.ShapeDtypeStruct((C, L), jnp.float32),
        mesh=mesh,
        scratch_types=[
            pltpu.VMEM((LW,), jnp.int32),
            pltpu.VMEM((LW,), jnp.int32),
        ] + [
            pltpu.VMEM((ROWS_PER_W, W), jnp.float32)
            for _ in range(NB)
        ] + [
            pltpu.VMEM((C, LW), jnp.float32),
            pltpu.SemaphoreType.DMA,
        ],
        compiler_params=pltpu.CompilerParams(needs_layout_passes=False),
    )
    return kfn(xr, sidx)


def kernel(x):
    b, c, h, w = x.shape
    sidx = jnp.asarray(_SIDX)
    xr = x.reshape(C * H, W)
    out = _run(xr, sidx)
    return out.reshape(1, C, L)


# speedup vs baseline: 1.4589x; 1.0109x over previous
import jax
import jax.numpy as jnp
import numpy as np
from jax import lax
from jax.experimental import pallas as pl
from jax.experimental.pallas import tpu as pltpu
from jax.experimental.pallas import tpu_sc as plsc

C = 96
H = 512
W = 512
FH = 128
FW = 128
L = FH * FW
NW = 32
PR_PER_W = FH // NW
ROWS_PER_W = 4 * PR_PER_W
LW = PR_PER_W * FW
CG = 4
NB = 2 * CG

_SIDX = np.asarray(
    jax.random.randint(jax.random.key(42), (L,), 0, 16, jnp.int32))


def _slab_src(x_hbm, wid, c):
    return x_hbm.at[pl.ds(c * H + wid * ROWS_PER_W, ROWS_PER_W), :]


def _body(x_hbm, s_hbm, out_hbm, sbuf, ibuf, xbufs, obuf, load_sem,
          store_sem, sem_s):
    cid = lax.axis_index("c")
    sid = lax.axis_index("s")
    wid = sid * 2 + cid
    base_l = wid * LW

    pltpu.async_copy(s_hbm.at[pl.ds(base_l, LW)], sbuf, sem_s)
    for b in range(NB):
        pltpu.async_copy(_slab_src(x_hbm, wid, b), xbufs[b], load_sem)

    pltpu.make_async_copy(s_hbm.at[pl.ds(base_l, LW)], sbuf, sem_s).wait()
    lane4 = lax.iota(jnp.int32, 16) * 4
    for i in range(LW // 16):
        s = sbuf[pl.ds(i * 16, 16)]
        row = (s >> 2) + ((i * 16) // FW) * 4
        col = (s & 3) + lane4 + ((i * 16) % FW) * 4
        ibuf[pl.ds(i * 16, 16)] = (row << 9) | col

    def group(g, carry):
        for half in range(NB // CG):
            c0 = g * NB + half * CG
            for k in range(CG):
                pltpu.make_async_copy(
                    _slab_src(x_hbm, wid, c0 + k), xbufs[half * CG + k],
                    load_sem,
                ).wait()
            for i in range(LW // 16):
                p = ibuf[pl.ds(i * 16, 16)]
                row = p >> 9
                col = p & (W - 1)
                for k in range(CG):
                    obuf[c0 + k, pl.ds(i * 16, 16)] = plsc.load_gather(
                        xbufs[half * CG + k], [row, col]
                    )

            @pl.when(c0 + NB + CG <= C)
            def _():
                for k in range(CG):
                    pltpu.async_copy(
                        _slab_src(x_hbm, wid, c0 + NB + k),
                        xbufs[half * CG + k], load_sem,
                    )

            pltpu.async_copy(
                obuf.at[pl.ds(c0, CG), :],
                out_hbm.at[pl.ds(c0, CG), pl.ds(base_l, LW)],
                store_sem,
            )

        return carry

    lax.fori_loop(0, C // NB, group, 0)
    for g in range(C // CG):
        pltpu.make_async_copy(
            obuf.at[pl.ds(g * CG, CG), :],
            out_hbm.at[pl.ds(g * CG, CG), pl.ds(base_l, LW)],
            store_sem,
        ).wait()


def _body_wrap(x_hbm, s_hbm, out_hbm, sbuf, ibuf, *rest):
    xbufs = rest[:NB]
    obuf, load_sem, store_sem, sem_s = rest[NB:]
    _body(x_hbm, s_hbm, out_hbm, sbuf, ibuf, xbufs, obuf, load_sem,
          store_sem, sem_s)


@jax.jit
def _run(xr, sidx):
    mesh = plsc.VectorSubcoreMesh(core_axis_name="c", subcore_axis_name="s")
    kfn = pl.kernel(
        _body_wrap,
        out_type=jax.ShapeDtypeStruct((C, L), jnp.float32),
        mesh=mesh,
        scratch_types=[
            pltpu.VMEM((LW,), jnp.int32),
            pltpu.VMEM((LW,), jnp.int32),
        ] + [
            pltpu.VMEM((ROWS_PER_W, W), jnp.float32)
            for _ in range(NB)
        ] + [
            pltpu.VMEM((C, LW), jnp.float32),
            pltpu.SemaphoreType.DMA,
            pltpu.SemaphoreType.DMA,
            pltpu.SemaphoreType.DMA,
        ],
        compiler_params=pltpu.CompilerParams(needs_layout_passes=False),
    )
    return kfn(xr, sidx)


def kernel(x):
    b, c, h, w = x.shape
    sidx = jnp.asarray(_SIDX)
    xr = x.reshape(C * H, W)
    out = _run(xr, sidx)
    return out.reshape(1, C, L)


# speedup vs baseline: 1.4614x; 1.0018x over previous
import jax
import jax.numpy as jnp
import numpy as np
from jax import lax
from jax.experimental import pallas as pl
from jax.experimental.pallas import tpu as pltpu
from jax.experimental.pallas import tpu_sc as plsc

C = 96
H = 512
W = 512
FH = 128
FW = 128
L = FH * FW
NW = 32
PR_PER_W = FH // NW
ROWS_PER_W = 4 * PR_PER_W
LW = PR_PER_W * FW
CG = 4
NB = 2 * CG

_SIDX = np.asarray(
    jax.random.randint(jax.random.key(42), (L,), 0, 16, jnp.int32))


def _slab_src(x_hbm, wid, c):
    return x_hbm.at[pl.ds(c * H + wid * ROWS_PER_W, ROWS_PER_W), :]


def _body(x_hbm, s_hbm, out_hbm, sbuf, ibuf, xbufs, obuf, load_sem,
          store_sem, sem_s):
    cid = lax.axis_index("c")
    sid = lax.axis_index("s")
    wid = sid * 2 + cid
    base_l = wid * LW

    pltpu.async_copy(s_hbm.at[pl.ds(base_l, LW)], sbuf, sem_s)
    for b in range(NB):
        pltpu.async_copy(_slab_src(x_hbm, wid, b), xbufs[b], load_sem)

    pltpu.make_async_copy(s_hbm.at[pl.ds(base_l, LW)], sbuf, sem_s).wait()
    lane4 = lax.iota(jnp.int32, 16) * 4
    for i in range(LW // 16):
        s = sbuf[pl.ds(i * 16, 16)]
        row = (s >> 2) + ((i * 16) // FW) * 4
        col = (s & 3) + lane4 + ((i * 16) % FW) * 4
        ibuf[pl.ds(i * 16, 16)] = (row << 9) | col

    def group(g, carry):
        for half in range(NB // CG):
            c0 = g * NB + half * CG
            for k in range(CG):
                pltpu.make_async_copy(
                    _slab_src(x_hbm, wid, c0 + k), xbufs[half * CG + k],
                    load_sem,
                ).wait()
            for i in range(LW // 16):
                p = ibuf[pl.ds(i * 16, 16)]
                row = p >> 9
                col = p & (W - 1)
                for k in range(CG):
                    obuf[c0 + k, pl.ds(i * 16, 16)] = plsc.load_gather(
                        xbufs[half * CG + k], [row, col]
                    )

            @pl.when(c0 + NB + CG <= C)
            def _():
                for k in range(CG):
                    pltpu.async_copy(
                        _slab_src(x_hbm, wid, c0 + NB + k),
                        xbufs[half * CG + k], load_sem,
                    )

            pltpu.async_copy(
                obuf.at[pl.ds(c0, CG), :],
                out_hbm.at[pl.ds(c0, CG), pl.ds(base_l, LW)],
                store_sem,
            )

        return carry

    lax.fori_loop(0, C // NB, group, 0)
    for g in range(C // CG):
        pltpu.make_async_copy(
            obuf.at[pl.ds(g * CG, CG), :],
            out_hbm.at[pl.ds(g * CG, CG), pl.ds(base_l, LW)],
            store_sem,
        ).wait()


def _body_wrap(x_hbm, s_hbm, out_hbm, sbuf, ibuf, *rest):
    xbufs = rest[:NB]
    obuf, load_sem, store_sem, sem_s = rest[NB:]
    _body(x_hbm, s_hbm, out_hbm, sbuf, ibuf, xbufs, obuf, load_sem,
          store_sem, sem_s)


@jax.jit
def _run(xr, sidx):
    mesh = plsc.VectorSubcoreMesh(core_axis_name="c", subcore_axis_name="s")
    kfn = pl.kernel(
        _body_wrap,
        out_type=jax.ShapeDtypeStruct((C, L), jnp.float32),
        mesh=mesh,
        scratch_types=[
            pltpu.VMEM((LW,), jnp.int32),
            pltpu.VMEM((LW,), jnp.int32),
        ] + [
            pltpu.VMEM((ROWS_PER_W, W), jnp.float32)
            for _ in range(NB)
        ] + [
            pltpu.VMEM((C, LW), jnp.float32),
            pltpu.SemaphoreType.DMA,
            pltpu.SemaphoreType.DMA,
            pltpu.SemaphoreType.DMA,
        ],
        compiler_params=pltpu.CompilerParams(
            needs_layout_passes=False, skip_device_barrier=True
        ),
    )
    return kfn(xr, sidx)


def kernel(x):
    b, c, h, w = x.shape
    sidx = jnp.asarray(_SIDX)
    xr = x.reshape(C * H, W)
    out = _run(xr, sidx)
    return out.reshape(1, C, L)


# speedup vs baseline: 1.5632x; 1.0697x over previous
import jax
import jax.numpy as jnp
import numpy as np
from jax import lax
from jax.experimental import pallas as pl
from jax.experimental.pallas import tpu as pltpu
from jax.experimental.pallas import tpu_sc as plsc

C = 96
H = 512
W = 512
FH = 128
FW = 128
L = FH * FW
NW = 32
PR_PER_W = FH // NW
ROWS_PER_W = 4 * PR_PER_W
LW = PR_PER_W * FW
CG = 4
NB = 2 * CG

with jax.default_device(jax.devices("cpu")[0]):
    _SIDX = np.asarray(
        jax.random.randint(jax.random.key(42), (L,), 0, 16, jnp.int32))


def _slab_src(x_hbm, wid, c):
    return x_hbm.at[pl.ds(c * H + wid * ROWS_PER_W, ROWS_PER_W), :]


def _body(x_hbm, s_hbm, out_hbm, sbuf, ibuf, xbufs, obuf, load_sem,
          store_sem, sem_s):
    cid = lax.axis_index("c")
    sid = lax.axis_index("s")
    wid = sid * 2 + cid
    base_l = wid * LW

    pltpu.async_copy(s_hbm.at[pl.ds(base_l, LW)], sbuf, sem_s)
    for b in range(NB):
        pltpu.async_copy(_slab_src(x_hbm, wid, b), xbufs[b], load_sem)

    pltpu.make_async_copy(s_hbm.at[pl.ds(base_l, LW)], sbuf, sem_s).wait()
    lane4 = lax.iota(jnp.int32, 16) * 4

    @plsc.parallel_loop(0, LW // 16, unroll=4)
    def _prolog(i):
        s = sbuf[pl.ds(i * 16, 16)]
        row = (s >> 2) + (i >> 3) * 4
        col = (s & 3) + lane4 + (i & 7) * 64
        ibuf[pl.ds(i * 16, 16)] = (row << 9) | col

    def group(g, carry):
        for half in range(NB // CG):
            c0 = g * NB + half * CG
            for k in range(CG):
                pltpu.make_async_copy(
                    _slab_src(x_hbm, wid, c0 + k), xbufs[half * CG + k],
                    load_sem,
                ).wait()
            @plsc.parallel_loop(0, LW // 16, unroll=4)
            def _sweep(i):
                p = ibuf[pl.ds(i * 16, 16)]
                row = p >> 9
                col = p & (W - 1)
                for k in range(CG):
                    obuf[c0 + k, pl.ds(i * 16, 16)] = plsc.load_gather(
                        xbufs[half * CG + k], [row, col]
                    )

            @pl.when(c0 + NB + CG <= C)
            def _():
                for k in range(CG):
                    pltpu.async_copy(
                        _slab_src(x_hbm, wid, c0 + NB + k),
                        xbufs[half * CG + k], load_sem,
                    )

            pltpu.async_copy(
                obuf.at[pl.ds(c0, CG), :],
                out_hbm.at[pl.ds(c0, CG), pl.ds(base_l, LW)],
                store_sem,
            )

        return carry

    lax.fori_loop(0, C // NB, group, 0)
    for g in range(C // CG):
        pltpu.make_async_copy(
            obuf.at[pl.ds(g * CG, CG), :],
            out_hbm.at[pl.ds(g * CG, CG), pl.ds(base_l, LW)],
            store_sem,
        ).wait()


def _body_wrap(x_hbm, s_hbm, out_hbm, sbuf, ibuf, *rest):
    xbufs = rest[:NB]
    obuf, load_sem, store_sem, sem_s = rest[NB:]
    _body(x_hbm, s_hbm, out_hbm, sbuf, ibuf, xbufs, obuf, load_sem,
          store_sem, sem_s)


@jax.jit
def _run(xr, sidx):
    mesh = plsc.VectorSubcoreMesh(core_axis_name="c", subcore_axis_name="s")
    kfn = pl.kernel(
        _body_wrap,
        out_type=jax.ShapeDtypeStruct((C, L), jnp.float32),
        mesh=mesh,
        scratch_types=[
            pltpu.VMEM((LW,), jnp.int32),
            pltpu.VMEM((LW,), jnp.int32),
        ] + [
            pltpu.VMEM((ROWS_PER_W, W), jnp.float32)
            for _ in range(NB)
        ] + [
            pltpu.VMEM((C, LW), jnp.float32),
            pltpu.SemaphoreType.DMA,
            pltpu.SemaphoreType.DMA,
            pltpu.SemaphoreType.DMA,
        ],
        compiler_params=pltpu.CompilerParams(needs_layout_passes=False),
    )
    return kfn(xr, sidx)


def kernel(x):
    b, c, h, w = x.shape
    sidx = jnp.asarray(_SIDX)
    xr = x.reshape(C * H, W)
    out = _run(xr, sidx)
    return out.reshape(1, C, L)
